# Initial kernel scaffold; baseline (speedup 1.0000x reference)
import jax
import jax.numpy as jnp
from jax.experimental import pallas as pl


def kernel(logits, scores, beam_size):
    raise NotImplementedError("write your pallas kernel here")


# speedup vs baseline: 71.5638x; 71.5638x over previous
import functools

import jax
import jax.numpy as jnp
from jax.experimental import pallas as pl

ROWS = 1024
V = 8192
RBLK = 128
KTOP = 4


def _stage1_body(x_ref, topv_ref, topi_ref):
    x = x_ref[...]
    iota = jax.lax.broadcasted_iota(jnp.int32, (RBLK, V), 1)
    cur = x
    vals = []
    idxs = []
    for k in range(KTOP):
        mk = jnp.max(cur, axis=1, keepdims=True)
        eq = cur == mk
        ik = jnp.min(jnp.where(eq, iota, V), axis=1, keepdims=True)
        vals.append(mk)
        idxs.append(ik)
        if k < KTOP - 1:
            cur = jnp.where(iota == ik, -jnp.inf, cur)
    m1 = vals[0]
    lse = m1 + jnp.log(jnp.sum(jnp.exp(x - m1), axis=1, keepdims=True))
    topv_ref[...] = jnp.concatenate(vals, axis=1) - lse
    topi_ref[...] = jnp.concatenate(idxs, axis=1)


def _stage2_body(tv_ref, gi_ref, sc_ref, best_ref, g0_ref, g1_ref, g2_ref, g3_ref):
    tv = tv_ref[...]
    gi = gi_ref[...]
    sc = sc_ref[...]
    cand = jnp.sum(tv, axis=-1) + sc
    iota = jax.lax.broadcasted_iota(jnp.int32, (32, 16), 1)
    cur = cand
    best_cols = []
    gen_refs = (g0_ref, g1_ref, g2_ref, g3_ref)
    for j in range(KTOP):
        mj = jnp.max(cur, axis=1, keepdims=True)
        eq = cur == mj
        ij = jnp.min(jnp.where(eq, iota, 16), axis=1, keepdims=True)
        cur = jnp.where(iota == ij, -jnp.inf, cur)
        best_cols.append(mj)
        acc = jnp.zeros((32, 8), jnp.int32)
        for r in range(16):
            acc = acc + jnp.where(ij == r, gi[r], 0)
        gen_refs[j][...] = acc
    best_ref[...] = jnp.concatenate(best_cols, axis=1)


@jax.jit
def _run(logits, scores):
    x = logits.reshape(ROWS, V)
    topv, topi = pl.pallas_call(
        _stage1_body,
        grid=(ROWS // RBLK,),
        in_specs=[pl.BlockSpec((RBLK, V), lambda i: (i, 0))],
        out_specs=[
            pl.BlockSpec((RBLK, KTOP), lambda i: (i, 0)),
            pl.BlockSpec((RBLK, KTOP), lambda i: (i, 0)),
        ],
        out_shape=[
            jax.ShapeDtypeStruct((ROWS, KTOP), jnp.float32),
            jax.ShapeDtypeStruct((ROWS, KTOP), jnp.int32),
        ],
    )(x)

    tv4 = topv.reshape(32, 4, 8, KTOP).transpose(0, 1, 3, 2).reshape(32, 16, 8)
    gi4 = topi.reshape(32, 4, 8, KTOP).transpose(1, 3, 0, 2).reshape(16, 32, 8)
    sc16 = jnp.broadcast_to(scores[:, :, None], (4, 32, KTOP))
    sc16 = sc16.transpose(1, 0, 2).reshape(32, 16)

    best_t, g0, g1, g2, g3 = pl.pallas_call(
        _stage2_body,
        out_shape=[
            jax.ShapeDtypeStruct((32, KTOP), jnp.float32),
            jax.ShapeDtypeStruct((32, 8), jnp.int32),
            jax.ShapeDtypeStruct((32, 8), jnp.int32),
            jax.ShapeDtypeStruct((32, 8), jnp.int32),
            jax.ShapeDtypeStruct((32, 8), jnp.int32),
        ],
    )(tv4, gi4, sc16)
    best = best_t.T
    gen = jnp.stack([g0, g1, g2, g3], axis=0)
    return best, gen


def kernel(logits, scores, beam_size):
    del beam_size
    return _run(logits, scores)
